# baseline (speedup 1.0000x reference)
import functools

import jax
import jax.numpy as jnp
from jax import lax
from jax.experimental import pallas as pl
from jax.experimental.pallas import tpu as pltpu
from jax.experimental.pallas import tpu_sc as plsc

N0, N1, N2 = 10000, 2500, 625
E0, E1, E2 = 320000, 80000, 20000
D = 128
NC, NS = 2, 16
NT = NC * NS
CH = 128

N0P, N1P, N2P = 10240, 2560, 640


def _pad_amount(e, m):
    return (m - e % m) % m


def _chunks(total, ch):
    out = []
    off = 0
    while off < total:
        sz = min(ch, total - off)
        out.append((off, sz))
        off += sz
    return out


def _sc_mesh():
    return plsc.VectorSubcoreMesh(core_axis_name="c", subcore_axis_name="s",
                                  num_cores=NC, num_subcores=NS)


@functools.partial(jax.jit, static_argnames=("npad", "width", "constant_rows"))
def _sc_scatter_partials(x, src, dst, npad, width, constant_rows=False):
    nch = dst.shape[0] // (NT * CH)
    sb = 4 if nch % 4 == 0 else 2
    assert nch % sb == 0 and sb % 2 == 0
    ngroups = nch // sb
    rpt = npad // NS
    zero_chunks = _chunks(rpt, CH)

    scratch = [
        pltpu.VMEM_SHARED((npad, width), jnp.float32),
        pltpu.VMEM((CH, width), jnp.float32),
        pltpu.VMEM((CH, width), jnp.float32),
        pltpu.VMEM((CH,), jnp.int32),
        pltpu.VMEM((CH,), jnp.int32),
        pltpu.VMEM((CH,), jnp.int32),
        pltpu.VMEM((CH,), jnp.int32),
        pltpu.SemaphoreType.DMA,
        pltpu.SemaphoreType.DMA,
        pltpu.SemaphoreType.DMA,
        pltpu.SemaphoreType.DMA,
        pltpu.SemaphoreType.DMA,
        pltpu.SemaphoreType.DMA,
    ]

    @functools.partial(
        pl.kernel,
        out_type=jax.ShapeDtypeStruct((NC, npad, width), jnp.float32),
        mesh=_sc_mesh(),
        scratch_types=scratch,
    )
    def k(x_h, src_h, dst_h, z_h, out_h, acc, rows_a, rows_b,
          sbuf_a, sbuf_b, dst_a, dst_b,
          sem_a, sem_b, ssem_a, ssem_b, dsem_a, dsem_b):
        c = lax.axis_index("c")
        s = lax.axis_index("s")
        tid = c * NS + s
        pltpu.sync_copy(z_h, rows_a)
        for off, sz in zero_chunks:
            if sz == CH:
                pltpu.sync_copy(rows_a, acc.at[pl.ds(s * rpt + off, CH)])
            else:
                pltpu.sync_copy(rows_a.at[pl.ds(0, sz)],
                                acc.at[pl.ds(s * rpt + off, sz)])
        if constant_rows:
            pltpu.sync_copy(x_h, rows_a)
        plsc.subcore_barrier()

        dbuf = (dst_a, dst_b)
        dsem = (dsem_a, dsem_b)
        gbuf = (rows_a, rows_b)
        gsem = (sem_a, sem_b)
        ept = nch * CH

        if constant_rows:
            @pl.loop(0, ngroups)
            def _(g):
                eb = tid * ept + g * sb * CH
                dd = {}

                def dstart(k):
                    dd[k] = pltpu.async_copy(
                        dst_h.at[pl.ds(eb + k * CH, CH)], dbuf[k % 2],
                        dsem[k % 2])

                dstart(0)
                dstart(1)
                for k in range(sb):
                    dd[k].wait()
                    pltpu.sync_copy(rows_a, acc.at[dbuf[k % 2]], add=True)
                    if k + 2 < sb:
                        dstart(k + 2)
        else:
            sbuf = (sbuf_a, sbuf_b)
            ssem = (ssem_a, ssem_b)

            @pl.loop(0, ngroups)
            def _(g):
                eb = tid * ept + g * sb * CH
                sd, dd, gd = {}, {}, {}

                def istart(k):
                    sd[k] = pltpu.async_copy(
                        src_h.at[pl.ds(eb + k * CH, CH)], sbuf[k % 2],
                        ssem[k % 2])
                    dd[k] = pltpu.async_copy(
                        dst_h.at[pl.ds(eb + k * CH, CH)], dbuf[k % 2],
                        dsem[k % 2])

                istart(0)
                istart(1)
                for k in range(sb):
                    sd[k].wait()
                    dd[k].wait()
                    pltpu.async_copy(x_h.at[sbuf[k % 2]], gbuf[k % 2],
                                     gsem[k % 2]).wait()
                    pltpu.sync_copy(gbuf[k % 2], acc.at[dbuf[k % 2]], add=True)
                    if k + 2 < sb:
                        istart(k + 2)
        plsc.subcore_barrier()
        pltpu.sync_copy(acc.at[pl.ds(s * rpt, rpt)],
                        out_h.at[c, pl.ds(s * rpt, rpt)])

    z = jnp.zeros((CH, width), jnp.float32)
    return k(x, src, dst, z)


@functools.partial(jax.jit, static_argnames=("npad",))
def _sc_count(dst, npad):
    ones = jnp.ones((CH, 128), jnp.float32)
    return _sc_scatter_partials(ones, dst, dst, npad, 128, constant_rows=True)


@functools.partial(jax.jit, static_argnames=())
def _sc_take(x, idx):
    mp = idx.shape[0]
    rpt = mp // NT
    chunk_list = _chunks(rpt, CH)
    rem = chunk_list[-1][1] if chunk_list[-1][1] != CH else 0

    scratch = [
        pltpu.VMEM((CH, D), jnp.float32),
        pltpu.VMEM((CH,), jnp.int32),
        pltpu.SemaphoreType.DMA,
    ]
    if rem:
        scratch += [pltpu.VMEM((rem, D), jnp.float32), pltpu.VMEM((rem,), jnp.int32)]

    @functools.partial(
        pl.kernel,
        out_type=jax.ShapeDtypeStruct((mp, D), jnp.float32),
        mesh=_sc_mesh(),
        scratch_types=scratch,
    )
    def k(x_h, idx_h, out_h, rows_v, idx_v, sem, *rest):
        c = lax.axis_index("c")
        s = lax.axis_index("s")
        base = (c * NS + s) * rpt
        nfull = sum(1 for _, sz in chunk_list if sz == CH)
        if nfull:
            @pl.loop(0, nfull)
            def _(i):
                b = base + i * CH
                pltpu.sync_copy(idx_h.at[pl.ds(b, CH)], idx_v)
                pltpu.async_copy(x_h.at[idx_v], rows_v, sem).wait()
                pltpu.sync_copy(rows_v, out_h.at[pl.ds(b, CH)])
        if rem:
            rows_r, idx_r = rest
            b = base + nfull * CH
            pltpu.sync_copy(idx_h.at[pl.ds(b, rem)], idx_r)
            pltpu.async_copy(x_h.at[idx_r], rows_r, sem).wait()
            pltpu.sync_copy(rows_r, out_h.at[pl.ds(b, rem)])

    return k(x, idx)


def _blk_rows(npad):
    return 512 if npad % 512 == 0 else 320


def _tc_conv(xs, ps, cnt, w, b, relu=True):
    P = len(xs)
    npad = xs[0].shape[0]
    R = _blk_rows(npad)
    grid = (npad // R,)

    def body(*refs):
        x_refs = refs[:P]
        p_refs = refs[P:2 * P]
        cnt_ref, w_ref, b_ref, o_ref = refs[2 * P:]
        deg = jnp.sum(cnt_ref[...], axis=0)
        invd = 1.0 / jnp.maximum(deg, 1.0)
        acc = None
        for p in range(P):
            mean = jnp.sum(p_refs[p][...], axis=0) * invd
            a = x_refs[p][...] + mean
            t = jnp.dot(a, w_ref[p * 128:(p + 1) * 128, :],
                        preferred_element_type=jnp.float32)
            acc = t if acc is None else acc + t
        acc = acc + b_ref[0:1, :]
        o_ref[...] = jnp.maximum(acc, 0.0) if relu else acc

    in_specs = (
        [pl.BlockSpec((R, 128), lambda i: (i, 0)) for _ in range(P)]
        + [pl.BlockSpec((2, R, 128), lambda i: (0, i, 0)) for _ in range(P)]
        + [pl.BlockSpec((2, R, 128), lambda i: (0, i, 0)),
           pl.BlockSpec((P * 128, 128), lambda i: (0, 0)),
           pl.BlockSpec((8, 128), lambda i: (0, 0))]
    )
    return pl.pallas_call(
        body,
        grid=grid,
        in_specs=in_specs,
        out_specs=pl.BlockSpec((R, 128), lambda i: (i, 0)),
        out_shape=jax.ShapeDtypeStruct((npad, 128), jnp.float32),
    )(*xs, *ps, cnt, w, b)


def _tc_scale(p, cnt):
    npad = p.shape[1]
    R = _blk_rows(npad)
    grid = (npad // R,)

    def body(p_ref, cnt_ref, o_ref):
        s = jnp.sum(p_ref[...], axis=0)
        deg = jnp.sum(cnt_ref[...], axis=0)
        o_ref[...] = s / jnp.maximum(deg, 1.0)

    return pl.pallas_call(
        body,
        grid=grid,
        in_specs=[pl.BlockSpec((2, R, 128), lambda i: (0, i, 0)),
                  pl.BlockSpec((2, R, 128), lambda i: (0, i, 0))],
        out_specs=pl.BlockSpec((R, 128), lambda i: (i, 0)),
        out_shape=jax.ShapeDtypeStruct((npad, 128), jnp.float32),
    )(p, cnt)


def _pad_edges(e, ep, dst_pad):
    n = e.shape[1]
    src = jnp.concatenate([e[0].astype(jnp.int32),
                           jnp.zeros((ep - n,), jnp.int32)])
    dst = jnp.concatenate([e[1].astype(jnp.int32),
                           jnp.full((ep - n,), dst_pad, jnp.int32)])
    return src, dst


def _pad_idx(idx, mp, fill):
    return jnp.concatenate([idx.astype(jnp.int32),
                            jnp.full((mp - idx.shape[0],), fill, jnp.int32)])


def _pad_w(w):
    din, dout = w.shape
    return jnp.pad(w, ((0, _pad_amount(din, 128)), (0, _pad_amount(dout, 128))))


def _pad_b(b):
    bp = jnp.pad(b, (0, _pad_amount(b.shape[0], 128)))
    return jnp.broadcast_to(bp[None, :], (8, bp.shape[0]))


def _edge_pad_len(e):
    return e + _pad_amount(e, NT * 2 * CH)


def _conv(x_parts, src, dst, cnt, w, b, npad):
    ps = [_sc_scatter_partials(xp, src, dst, npad, 128) for xp in x_parts]
    return _tc_conv(x_parts, ps, cnt, _pad_w(w), _pad_b(b))


def kernel(norm, geo, e0, e1, e2, cluster1, cluster2, Ws, bs):
    x = jnp.concatenate([norm, geo[:, None]], axis=1)
    x = jnp.pad(x, ((0, N0P - N0), (0, 128 - 4)))

    s0, d0 = _pad_edges(e0, _edge_pad_len(E0), N0)
    s1, d1 = _pad_edges(e1, _edge_pad_len(E1), N1)
    s2, d2 = _pad_edges(e2, _edge_pad_len(E2), N2)
    cl1 = _pad_idx(cluster1, N0P, 0)
    cl2 = _pad_idx(cluster2, N1P, 0)
    p1e = _edge_pad_len(N0)
    p2e = _edge_pad_len(N1)
    pool1_src = _pad_idx(jnp.arange(N0, dtype=jnp.int32), p1e, 0)
    pool1_dst = _pad_idx(cluster1, p1e, N1)
    pool2_src = _pad_idx(jnp.arange(N1, dtype=jnp.int32), p2e, 0)
    pool2_dst = _pad_idx(cluster2, p2e, N2)

    cnt_e0 = _sc_count(d0, N0P)
    cnt_e1 = _sc_count(d1, N1P)
    cnt_e2 = _sc_count(d2, N2P)
    cnt_c1 = _sc_count(pool1_dst, N1P)
    cnt_c2 = _sc_count(pool2_dst, N2P)

    x = _conv([x], s0, d0, cnt_e0, Ws[0], bs[0], N0P)
    x = _conv([x], s0, d0, cnt_e0, Ws[1], bs[1], N0P)
    copy0 = x
    p = _sc_scatter_partials(x, pool1_src, pool1_dst, N1P, 128)
    x = _tc_scale(p, cnt_c1)
    x = _conv([x], s1, d1, cnt_e1, Ws[2], bs[2], N1P)
    x = _conv([x], s1, d1, cnt_e1, Ws[3], bs[3], N1P)
    copy1 = x
    p = _sc_scatter_partials(x, pool2_src, pool2_dst, N2P, 128)
    x = _tc_scale(p, cnt_c2)
    x = _conv([x], s2, d2, cnt_e2, Ws[4], bs[4], N2P)
    x = _conv([x], s2, d2, cnt_e2, Ws[5], bs[5], N2P)

    up = _sc_take(x, cl2)
    x = _conv([up, copy1], s1, d1, cnt_e1, Ws[6], bs[6], N1P)
    for i in range(7, 10):
        x = _conv([x], s1, d1, cnt_e1, Ws[i], bs[i], N1P)

    up = _sc_take(x, cl1)
    x = _conv([up, copy0], s0, d0, cnt_e0, Ws[10], bs[10], N0P)
    x = _conv([x], s0, d0, cnt_e0, Ws[11], bs[11], N0P)
    x = _conv([x], s0, d0, cnt_e0, Ws[12], bs[12], N0P)
    x = _conv([x], s0, d0, cnt_e0, Ws[13], bs[13], N0P)

    return x[:N0, 0]


# speedup vs baseline: 2.3905x; 2.3905x over previous
import functools

import jax
import jax.numpy as jnp
from jax import lax
from jax.experimental import pallas as pl
from jax.experimental.pallas import tpu as pltpu
from jax.experimental.pallas import tpu_sc as plsc

N0, N1, N2 = 10000, 2500, 625
E0, E1, E2 = 320000, 80000, 20000
D = 128
NC, NS = 2, 16
NT = NC * NS
CH = 128

N0P, N1P, N2P = 10240, 2560, 640


def _pad_amount(e, m):
    return (m - e % m) % m


def _chunks(total, ch):
    out = []
    off = 0
    while off < total:
        sz = min(ch, total - off)
        out.append((off, sz))
        off += sz
    return out


def _sc_mesh():
    return plsc.VectorSubcoreMesh(core_axis_name="c", subcore_axis_name="s",
                                  num_cores=NC, num_subcores=NS)


@functools.partial(jax.jit, static_argnames=("npad", "width", "constant_rows"))
def _sc_scatter_partials(x, src, dst, npad, width, constant_rows=False):
    ep = dst.shape[0]
    ept = ep // NT
    chunk_list = _chunks(ept, CH)
    rpt = npad // NS
    zero_chunks = _chunks(rpt, CH)

    scratch = [
        pltpu.VMEM_SHARED((npad, width), jnp.float32),
        pltpu.VMEM((CH, width), jnp.float32),
        pltpu.VMEM((CH,), jnp.int32),
        pltpu.VMEM((CH,), jnp.int32),
        pltpu.SemaphoreType.DMA,
    ]
    rem = chunk_list[-1][1] if chunk_list[-1][1] != CH else 0
    if rem:
        scratch += [
            pltpu.VMEM((rem, width), jnp.float32),
            pltpu.VMEM((rem,), jnp.int32),
            pltpu.VMEM((rem,), jnp.int32),
        ]

    @functools.partial(
        pl.kernel,
        out_type=jax.ShapeDtypeStruct((NC, npad, width), jnp.float32),
        mesh=_sc_mesh(),
        scratch_types=scratch,
    )
    def k(x_h, src_h, dst_h, z_h, out_h, acc, rows_v, src_v, dst_v, sem, *rest):
        c = lax.axis_index("c")
        s = lax.axis_index("s")
        tid = c * NS + s
        pltpu.sync_copy(z_h, rows_v)
        for off, sz in zero_chunks:
            if sz == CH:
                pltpu.sync_copy(rows_v, acc.at[pl.ds(s * rpt + off, CH)])
            else:
                pltpu.sync_copy(rows_v.at[pl.ds(0, sz)],
                                acc.at[pl.ds(s * rpt + off, sz)])
        if constant_rows:
            pltpu.sync_copy(x_h, rows_v)
            if rem:
                pltpu.sync_copy(x_h.at[pl.ds(0, rem)], rest[0])
        plsc.subcore_barrier()

        base = tid * ept
        nfull = sum(1 for _, sz in chunk_list if sz == CH)
        if nfull:
            @pl.loop(0, nfull)
            def _(i):
                b = base + i * CH
                if not constant_rows:
                    pltpu.sync_copy(src_h.at[pl.ds(b, CH)], src_v)
                    pltpu.async_copy(x_h.at[src_v], rows_v, sem).wait()
                pltpu.sync_copy(dst_h.at[pl.ds(b, CH)], dst_v)
                pltpu.sync_copy(rows_v, acc.at[dst_v], add=True)
        if rem:
            rows_r, src_r, dst_r = rest
            b = base + nfull * CH
            if not constant_rows:
                pltpu.sync_copy(src_h.at[pl.ds(b, rem)], src_r)
                pltpu.async_copy(x_h.at[src_r], rows_r, sem).wait()
            pltpu.sync_copy(dst_h.at[pl.ds(b, rem)], dst_r)
            pltpu.sync_copy(rows_r, acc.at[dst_r], add=True)
        plsc.subcore_barrier()
        pltpu.sync_copy(acc.at[pl.ds(s * rpt, rpt)],
                        out_h.at[c, pl.ds(s * rpt, rpt)])

    z = jnp.zeros((CH, width), jnp.float32)
    return k(x, src, dst, z)


@functools.partial(jax.jit, static_argnames=("npad",))
def _sc_count(dst, npad):
    ones = jnp.ones((CH, 128), jnp.float32)
    return _sc_scatter_partials(ones, dst, dst, npad, 128, constant_rows=True)


@functools.partial(jax.jit, static_argnames=())
def _sc_take(x, idx):
    mp = idx.shape[0]
    rpt = mp // NT
    chunk_list = _chunks(rpt, CH)
    rem = chunk_list[-1][1] if chunk_list[-1][1] != CH else 0

    scratch = [
        pltpu.VMEM((CH, D), jnp.float32),
        pltpu.VMEM((CH,), jnp.int32),
        pltpu.SemaphoreType.DMA,
    ]
    if rem:
        scratch += [pltpu.VMEM((rem, D), jnp.float32), pltpu.VMEM((rem,), jnp.int32)]

    @functools.partial(
        pl.kernel,
        out_type=jax.ShapeDtypeStruct((mp, D), jnp.float32),
        mesh=_sc_mesh(),
        scratch_types=scratch,
    )
    def k(x_h, idx_h, out_h, rows_v, idx_v, sem, *rest):
        c = lax.axis_index("c")
        s = lax.axis_index("s")
        base = (c * NS + s) * rpt
        nfull = sum(1 for _, sz in chunk_list if sz == CH)
        if nfull:
            @pl.loop(0, nfull)
            def _(i):
                b = base + i * CH
                pltpu.sync_copy(idx_h.at[pl.ds(b, CH)], idx_v)
                pltpu.async_copy(x_h.at[idx_v], rows_v, sem).wait()
                pltpu.sync_copy(rows_v, out_h.at[pl.ds(b, CH)])
        if rem:
            rows_r, idx_r = rest
            b = base + nfull * CH
            pltpu.sync_copy(idx_h.at[pl.ds(b, rem)], idx_r)
            pltpu.async_copy(x_h.at[idx_r], rows_r, sem).wait()
            pltpu.sync_copy(rows_r, out_h.at[pl.ds(b, rem)])

    return k(x, idx)


def _blk_rows(npad):
    return 512 if npad % 512 == 0 else 320


def _tc_conv(xs, ps, cnt, w, b, relu=True):
    P = len(xs)
    npad = xs[0].shape[0]
    R = _blk_rows(npad)
    grid = (npad // R,)

    def body(*refs):
        x_refs = refs[:P]
        p_refs = refs[P:2 * P]
        cnt_ref, w_ref, b_ref, o_ref = refs[2 * P:]
        deg = jnp.sum(cnt_ref[...], axis=0)
        invd = 1.0 / jnp.maximum(deg, 1.0)
        acc = None
        for p in range(P):
            mean = jnp.sum(p_refs[p][...], axis=0) * invd
            a = x_refs[p][...] + mean
            t = jnp.dot(a, w_ref[p * 128:(p + 1) * 128, :],
                        preferred_element_type=jnp.float32)
            acc = t if acc is None else acc + t
        acc = acc + b_ref[0:1, :]
        o_ref[...] = jnp.maximum(acc, 0.0) if relu else acc

    in_specs = (
        [pl.BlockSpec((R, 128), lambda i: (i, 0)) for _ in range(P)]
        + [pl.BlockSpec((2, R, 128), lambda i: (0, i, 0)) for _ in range(P)]
        + [pl.BlockSpec((2, R, 128), lambda i: (0, i, 0)),
           pl.BlockSpec((P * 128, 128), lambda i: (0, 0)),
           pl.BlockSpec((8, 128), lambda i: (0, 0))]
    )
    return pl.pallas_call(
        body,
        grid=grid,
        in_specs=in_specs,
        out_specs=pl.BlockSpec((R, 128), lambda i: (i, 0)),
        out_shape=jax.ShapeDtypeStruct((npad, 128), jnp.float32),
    )(*xs, *ps, cnt, w, b)


def _tc_scale(p, cnt):
    npad = p.shape[1]
    R = _blk_rows(npad)
    grid = (npad // R,)

    def body(p_ref, cnt_ref, o_ref):
        s = jnp.sum(p_ref[...], axis=0)
        deg = jnp.sum(cnt_ref[...], axis=0)
        o_ref[...] = s / jnp.maximum(deg, 1.0)

    return pl.pallas_call(
        body,
        grid=grid,
        in_specs=[pl.BlockSpec((2, R, 128), lambda i: (0, i, 0)),
                  pl.BlockSpec((2, R, 128), lambda i: (0, i, 0))],
        out_specs=pl.BlockSpec((R, 128), lambda i: (i, 0)),
        out_shape=jax.ShapeDtypeStruct((npad, 128), jnp.float32),
    )(p, cnt)


def _pad_edges(e, ep, dst_pad):
    n = e.shape[1]
    src = jnp.concatenate([e[0].astype(jnp.int32),
                           jnp.zeros((ep - n,), jnp.int32)])
    dst = jnp.concatenate([e[1].astype(jnp.int32),
                           jnp.full((ep - n,), dst_pad, jnp.int32)])
    return src, dst


def _pad_idx(idx, mp, fill):
    return jnp.concatenate([idx.astype(jnp.int32),
                            jnp.full((mp - idx.shape[0],), fill, jnp.int32)])


def _pad_w(w):
    din, dout = w.shape
    return jnp.pad(w, ((0, _pad_amount(din, 128)), (0, _pad_amount(dout, 128))))


def _pad_b(b):
    bp = jnp.pad(b, (0, _pad_amount(b.shape[0], 128)))
    return jnp.broadcast_to(bp[None, :], (8, bp.shape[0]))


def _edge_pad_len(e):
    return e + _pad_amount(e, NT * 8)


def _conv(x_parts, src, dst, cnt, w, b, npad):
    ps = [_sc_scatter_partials(xp, src, dst, npad, 128) for xp in x_parts]
    return _tc_conv(x_parts, ps, cnt, _pad_w(w), _pad_b(b))


def kernel(norm, geo, e0, e1, e2, cluster1, cluster2, Ws, bs):
    x = jnp.concatenate([norm, geo[:, None]], axis=1)
    x = jnp.pad(x, ((0, N0P - N0), (0, 128 - 4)))

    s0, d0 = _pad_edges(e0, _edge_pad_len(E0), N0)
    s1, d1 = _pad_edges(e1, _edge_pad_len(E1), N1)
    s2, d2 = _pad_edges(e2, _edge_pad_len(E2), N2)
    cl1 = _pad_idx(cluster1, N0P, 0)
    cl2 = _pad_idx(cluster2, N1P, 0)
    p1e = _edge_pad_len(N0)
    p2e = _edge_pad_len(N1)
    pool1_src = _pad_idx(jnp.arange(N0, dtype=jnp.int32), p1e, 0)
    pool1_dst = _pad_idx(cluster1, p1e, N1)
    pool2_src = _pad_idx(jnp.arange(N1, dtype=jnp.int32), p2e, 0)
    pool2_dst = _pad_idx(cluster2, p2e, N2)

    cnt_e0 = _sc_count(d0, N0P)
    cnt_e1 = _sc_count(d1, N1P)
    cnt_e2 = _sc_count(d2, N2P)
    cnt_c1 = _sc_count(pool1_dst, N1P)
    cnt_c2 = _sc_count(pool2_dst, N2P)

    x = _conv([x], s0, d0, cnt_e0, Ws[0], bs[0], N0P)
    x = _conv([x], s0, d0, cnt_e0, Ws[1], bs[1], N0P)
    copy0 = x
    p = _sc_scatter_partials(x, pool1_src, pool1_dst, N1P, 128)
    x = _tc_scale(p, cnt_c1)
    x = _conv([x], s1, d1, cnt_e1, Ws[2], bs[2], N1P)
    x = _conv([x], s1, d1, cnt_e1, Ws[3], bs[3], N1P)
    copy1 = x
    p = _sc_scatter_partials(x, pool2_src, pool2_dst, N2P, 128)
    x = _tc_scale(p, cnt_c2)
    x = _conv([x], s2, d2, cnt_e2, Ws[4], bs[4], N2P)
    x = _conv([x], s2, d2, cnt_e2, Ws[5], bs[5], N2P)

    up = _sc_take(x, cl2)
    x = _conv([up, copy1], s1, d1, cnt_e1, Ws[6], bs[6], N1P)
    for i in range(7, 10):
        x = _conv([x], s1, d1, cnt_e1, Ws[i], bs[i], N1P)

    up = _sc_take(x, cl1)
    x = _conv([up, copy0], s0, d0, cnt_e0, Ws[10], bs[10], N0P)
    x = _conv([x], s0, d0, cnt_e0, Ws[11], bs[11], N0P)
    x = _conv([x], s0, d0, cnt_e0, Ws[12], bs[12], N0P)
    x = _conv([x], s0, d0, cnt_e0, Ws[13], bs[13], N0P)

    return x[:N0, 0]


# speedup vs baseline: 2.6981x; 1.1287x over previous
import functools

import jax
import jax.numpy as jnp
from jax import lax
from jax.experimental import pallas as pl
from jax.experimental.pallas import tpu as pltpu
from jax.experimental.pallas import tpu_sc as plsc

N0, N1, N2 = 10000, 2500, 625
E0, E1, E2 = 320000, 80000, 20000
D = 128
NC, NS = 2, 16
NT = NC * NS
CH = 128

N0P, N1P, N2P = 10240, 2560, 640


def _pad_amount(e, m):
    return (m - e % m) % m


def _chunks(total, ch):
    out = []
    off = 0
    while off < total:
        sz = min(ch, total - off)
        out.append((off, sz))
        off += sz
    return out


def _sc_mesh():
    return plsc.VectorSubcoreMesh(core_axis_name="c", subcore_axis_name="s",
                                  num_cores=NC, num_subcores=NS)


@functools.partial(jax.jit, static_argnames=("npad", "width", "constant_rows"))
def _sc_scatter_partials(x, src, dst, npad, width, constant_rows=False):
    ep = dst.shape[0]
    ept = ep // NT
    chunk_list = _chunks(ept, CH)
    rpt = npad // NS
    zero_chunks = _chunks(rpt, CH)

    scratch = [
        pltpu.VMEM_SHARED((npad, width), jnp.float32),
        pltpu.VMEM((CH, width), jnp.float32),
        pltpu.VMEM((CH,), jnp.int32),
        pltpu.VMEM((CH,), jnp.int32),
        pltpu.SemaphoreType.DMA,
    ]
    rem = chunk_list[-1][1] if chunk_list[-1][1] != CH else 0
    if rem:
        scratch += [
            pltpu.VMEM((rem, width), jnp.float32),
            pltpu.VMEM((rem,), jnp.int32),
            pltpu.VMEM((rem,), jnp.int32),
        ]

    @functools.partial(
        pl.kernel,
        out_type=jax.ShapeDtypeStruct((NC, npad, width), jnp.float32),
        mesh=_sc_mesh(),
        scratch_types=scratch,
    )
    def k(x_h, src_h, dst_h, z_h, out_h, acc, rows_v, src_v, dst_v, sem, *rest):
        c = lax.axis_index("c")
        s = lax.axis_index("s")
        tid = c * NS + s
        pltpu.sync_copy(z_h, rows_v)
        for off, sz in zero_chunks:
            if sz == CH:
                pltpu.sync_copy(rows_v, acc.at[pl.ds(s * rpt + off, CH)])
            else:
                pltpu.sync_copy(rows_v.at[pl.ds(0, sz)],
                                acc.at[pl.ds(s * rpt + off, sz)])
        if constant_rows:
            pltpu.sync_copy(x_h, rows_v)
            if rem:
                pltpu.sync_copy(x_h.at[pl.ds(0, rem)], rest[0])
        plsc.subcore_barrier()

        base = tid * ept
        nfull = sum(1 for _, sz in chunk_list if sz == CH)
        if nfull:
            @pl.loop(0, nfull)
            def _(i):
                b = base + i * CH
                if not constant_rows:
                    pltpu.sync_copy(src_h.at[pl.ds(b, CH)], src_v)
                    gd = pltpu.async_copy(x_h.at[src_v], rows_v, sem)
                    pltpu.sync_copy(dst_h.at[pl.ds(b, CH)], dst_v)
                    gd.wait()
                else:
                    pltpu.sync_copy(dst_h.at[pl.ds(b, CH)], dst_v)
                pltpu.sync_copy(rows_v, acc.at[dst_v], add=True)
        if rem:
            rows_r, src_r, dst_r = rest
            b = base + nfull * CH
            if not constant_rows:
                pltpu.sync_copy(src_h.at[pl.ds(b, rem)], src_r)
                pltpu.async_copy(x_h.at[src_r], rows_r, sem).wait()
            pltpu.sync_copy(dst_h.at[pl.ds(b, rem)], dst_r)
            pltpu.sync_copy(rows_r, acc.at[dst_r], add=True)
        plsc.subcore_barrier()
        pltpu.sync_copy(acc.at[pl.ds(s * rpt, rpt)],
                        out_h.at[c, pl.ds(s * rpt, rpt)])

    z = jnp.zeros((CH, width), jnp.float32)
    return k(x, src, dst, z)


@functools.partial(jax.jit, static_argnames=("npad",))
def _sc_count(dst, npad):
    ones = jnp.ones((CH, 128), jnp.float32)
    return _sc_scatter_partials(ones, dst, dst, npad, 128, constant_rows=True)


@functools.partial(jax.jit, static_argnames=())
def _sc_take(x, idx):
    mp = idx.shape[0]
    rpt = mp // NT
    chunk_list = _chunks(rpt, CH)
    rem = chunk_list[-1][1] if chunk_list[-1][1] != CH else 0

    scratch = [
        pltpu.VMEM((CH, D), jnp.float32),
        pltpu.VMEM((CH,), jnp.int32),
        pltpu.SemaphoreType.DMA,
    ]
    if rem:
        scratch += [pltpu.VMEM((rem, D), jnp.float32), pltpu.VMEM((rem,), jnp.int32)]

    @functools.partial(
        pl.kernel,
        out_type=jax.ShapeDtypeStruct((mp, D), jnp.float32),
        mesh=_sc_mesh(),
        scratch_types=scratch,
    )
    def k(x_h, idx_h, out_h, rows_v, idx_v, sem, *rest):
        c = lax.axis_index("c")
        s = lax.axis_index("s")
        base = (c * NS + s) * rpt
        nfull = sum(1 for _, sz in chunk_list if sz == CH)
        if nfull:
            @pl.loop(0, nfull)
            def _(i):
                b = base + i * CH
                pltpu.sync_copy(idx_h.at[pl.ds(b, CH)], idx_v)
                pltpu.async_copy(x_h.at[idx_v], rows_v, sem).wait()
                pltpu.sync_copy(rows_v, out_h.at[pl.ds(b, CH)])
        if rem:
            rows_r, idx_r = rest
            b = base + nfull * CH
            pltpu.sync_copy(idx_h.at[pl.ds(b, rem)], idx_r)
            pltpu.async_copy(x_h.at[idx_r], rows_r, sem).wait()
            pltpu.sync_copy(rows_r, out_h.at[pl.ds(b, rem)])

    return k(x, idx)


def _blk_rows(npad):
    return 512 if npad % 512 == 0 else 320


def _tc_conv(xs, ps, cnt, w, b, relu=True):
    P = len(xs)
    npad = xs[0].shape[0]
    R = _blk_rows(npad)
    grid = (npad // R,)

    def body(*refs):
        x_refs = refs[:P]
        p_refs = refs[P:2 * P]
        cnt_ref, w_ref, b_ref, o_ref = refs[2 * P:]
        deg = jnp.sum(cnt_ref[...], axis=0)
        invd = 1.0 / jnp.maximum(deg, 1.0)
        acc = None
        for p in range(P):
            mean = jnp.sum(p_refs[p][...], axis=0) * invd
            a = x_refs[p][...] + mean
            t = jnp.dot(a, w_ref[p * 128:(p + 1) * 128, :],
                        preferred_element_type=jnp.float32)
            acc = t if acc is None else acc + t
        acc = acc + b_ref[0:1, :]
        o_ref[...] = jnp.maximum(acc, 0.0) if relu else acc

    in_specs = (
        [pl.BlockSpec((R, 128), lambda i: (i, 0)) for _ in range(P)]
        + [pl.BlockSpec((2, R, 128), lambda i: (0, i, 0)) for _ in range(P)]
        + [pl.BlockSpec((2, R, 128), lambda i: (0, i, 0)),
           pl.BlockSpec((P * 128, 128), lambda i: (0, 0)),
           pl.BlockSpec((8, 128), lambda i: (0, 0))]
    )
    return pl.pallas_call(
        body,
        grid=grid,
        in_specs=in_specs,
        out_specs=pl.BlockSpec((R, 128), lambda i: (i, 0)),
        out_shape=jax.ShapeDtypeStruct((npad, 128), jnp.float32),
    )(*xs, *ps, cnt, w, b)


def _tc_scale(p, cnt):
    npad = p.shape[1]
    R = _blk_rows(npad)
    grid = (npad // R,)

    def body(p_ref, cnt_ref, o_ref):
        s = jnp.sum(p_ref[...], axis=0)
        deg = jnp.sum(cnt_ref[...], axis=0)
        o_ref[...] = s / jnp.maximum(deg, 1.0)

    return pl.pallas_call(
        body,
        grid=grid,
        in_specs=[pl.BlockSpec((2, R, 128), lambda i: (0, i, 0)),
                  pl.BlockSpec((2, R, 128), lambda i: (0, i, 0))],
        out_specs=pl.BlockSpec((R, 128), lambda i: (i, 0)),
        out_shape=jax.ShapeDtypeStruct((npad, 128), jnp.float32),
    )(p, cnt)


def _pad_edges(e, ep, dst_pad):
    n = e.shape[1]
    src = jnp.concatenate([e[0].astype(jnp.int32),
                           jnp.zeros((ep - n,), jnp.int32)])
    dst = jnp.concatenate([e[1].astype(jnp.int32),
                           jnp.full((ep - n,), dst_pad, jnp.int32)])
    return src, dst


def _pad_idx(idx, mp, fill):
    return jnp.concatenate([idx.astype(jnp.int32),
                            jnp.full((mp - idx.shape[0],), fill, jnp.int32)])


def _pad_w(w):
    din, dout = w.shape
    return jnp.pad(w, ((0, _pad_amount(din, 128)), (0, _pad_amount(dout, 128))))


def _pad_b(b):
    bp = jnp.pad(b, (0, _pad_amount(b.shape[0], 128)))
    return jnp.broadcast_to(bp[None, :], (8, bp.shape[0]))


def _edge_pad_len(e):
    return e + _pad_amount(e, NT * 8)


def _conv(x_parts, src, dst, cnt, w, b, npad):
    ps = [_sc_scatter_partials(xp, src, dst, npad, 128) for xp in x_parts]
    return _tc_conv(x_parts, ps, cnt, _pad_w(w), _pad_b(b))


def kernel(norm, geo, e0, e1, e2, cluster1, cluster2, Ws, bs):
    x = jnp.concatenate([norm, geo[:, None]], axis=1)
    x = jnp.pad(x, ((0, N0P - N0), (0, 128 - 4)))

    s0, d0 = _pad_edges(e0, _edge_pad_len(E0), N0)
    s1, d1 = _pad_edges(e1, _edge_pad_len(E1), N1)
    s2, d2 = _pad_edges(e2, _edge_pad_len(E2), N2)
    cl1 = _pad_idx(cluster1, N0P, 0)
    cl2 = _pad_idx(cluster2, N1P, 0)
    p1e = _edge_pad_len(N0)
    p2e = _edge_pad_len(N1)
    pool1_src = _pad_idx(jnp.arange(N0, dtype=jnp.int32), p1e, 0)
    pool1_dst = _pad_idx(cluster1, p1e, N1)
    pool2_src = _pad_idx(jnp.arange(N1, dtype=jnp.int32), p2e, 0)
    pool2_dst = _pad_idx(cluster2, p2e, N2)

    cnt_e0 = _sc_count(d0, N0P)
    cnt_e1 = _sc_count(d1, N1P)
    cnt_e2 = _sc_count(d2, N2P)
    cnt_c1 = _sc_count(pool1_dst, N1P)
    cnt_c2 = _sc_count(pool2_dst, N2P)

    x = _conv([x], s0, d0, cnt_e0, Ws[0], bs[0], N0P)
    x = _conv([x], s0, d0, cnt_e0, Ws[1], bs[1], N0P)
    copy0 = x
    p = _sc_scatter_partials(x, pool1_src, pool1_dst, N1P, 128)
    x = _tc_scale(p, cnt_c1)
    x = _conv([x], s1, d1, cnt_e1, Ws[2], bs[2], N1P)
    x = _conv([x], s1, d1, cnt_e1, Ws[3], bs[3], N1P)
    copy1 = x
    p = _sc_scatter_partials(x, pool2_src, pool2_dst, N2P, 128)
    x = _tc_scale(p, cnt_c2)
    x = _conv([x], s2, d2, cnt_e2, Ws[4], bs[4], N2P)
    x = _conv([x], s2, d2, cnt_e2, Ws[5], bs[5], N2P)

    up = _sc_take(x, cl2)
    x = _conv([up, copy1], s1, d1, cnt_e1, Ws[6], bs[6], N1P)
    for i in range(7, 10):
        x = _conv([x], s1, d1, cnt_e1, Ws[i], bs[i], N1P)

    up = _sc_take(x, cl1)
    x = _conv([up, copy0], s0, d0, cnt_e0, Ws[10], bs[10], N0P)
    x = _conv([x], s0, d0, cnt_e0, Ws[11], bs[11], N0P)
    x = _conv([x], s0, d0, cnt_e0, Ws[12], bs[12], N0P)
    x = _conv([x], s0, d0, cnt_e0, Ws[13], bs[13], N0P)

    return x[:N0, 0]


# speedup vs baseline: 3.0566x; 1.1329x over previous
import functools

import jax
import jax.numpy as jnp
from jax import lax
from jax.experimental import pallas as pl
from jax.experimental.pallas import tpu as pltpu
from jax.experimental.pallas import tpu_sc as plsc

N0, N1, N2 = 10000, 2500, 625
E0, E1, E2 = 320000, 80000, 20000
D = 128
NC, NS = 2, 16
NT = NC * NS
CH = 128

N0P, N1P, N2P = 10240, 2560, 640


def _pad_amount(e, m):
    return (m - e % m) % m


def _chunks(total, ch):
    out = []
    off = 0
    while off < total:
        sz = min(ch, total - off)
        out.append((off, sz))
        off += sz
    return out


def _sc_mesh():
    return plsc.VectorSubcoreMesh(core_axis_name="c", subcore_axis_name="s",
                                  num_cores=NC, num_subcores=NS)


@functools.partial(jax.jit, static_argnames=("npad", "width", "constant_rows"))
def _sc_scatter_partials(x, src, dst, npad, width, constant_rows=False):
    ep = dst.shape[0] - CH
    ept = ep // NT
    chunk_list = _chunks(ept, CH)
    rpt = npad // NS
    zero_chunks = _chunks(rpt, CH)

    scratch = [
        pltpu.VMEM_SHARED((npad, width), jnp.float32),
        pltpu.VMEM((CH, width), jnp.float32),
        pltpu.VMEM((CH,), jnp.int32),
        pltpu.VMEM((CH,), jnp.int32),
        pltpu.SemaphoreType.DMA,
        pltpu.SemaphoreType.DMA,
    ]
    rem = chunk_list[-1][1] if chunk_list[-1][1] != CH else 0
    if rem:
        scratch += [
            pltpu.VMEM((rem, width), jnp.float32),
            pltpu.VMEM((rem,), jnp.int32),
            pltpu.VMEM((rem,), jnp.int32),
        ]

    @functools.partial(
        pl.kernel,
        out_type=jax.ShapeDtypeStruct((NC, npad, width), jnp.float32),
        mesh=_sc_mesh(),
        scratch_types=scratch,
    )
    def k(x_h, src_h, dst_h, z_h, out_h, acc, rows_v, src_v, dst_v, sem,
          sem2, *rest):
        c = lax.axis_index("c")
        s = lax.axis_index("s")
        tid = c * NS + s
        pltpu.sync_copy(z_h, rows_v)
        for off, sz in zero_chunks:
            if sz == CH:
                pltpu.sync_copy(rows_v, acc.at[pl.ds(s * rpt + off, CH)])
            else:
                pltpu.sync_copy(rows_v.at[pl.ds(0, sz)],
                                acc.at[pl.ds(s * rpt + off, sz)])
        if constant_rows:
            pltpu.sync_copy(x_h, rows_v)
            if rem:
                pltpu.sync_copy(x_h.at[pl.ds(0, rem)], rest[0])
        plsc.subcore_barrier()

        base = tid * ept
        nfull = sum(1 for _, sz in chunk_list if sz == CH)
        if nfull:
            if not constant_rows:
                pltpu.sync_copy(src_h.at[pl.ds(base, CH)], src_v)

            @pl.loop(0, nfull)
            def _(i):
                b = base + i * CH
                if not constant_rows:
                    gd = pltpu.async_copy(x_h.at[src_v], rows_v, sem)
                    pltpu.sync_copy(dst_h.at[pl.ds(b, CH)], dst_v)
                    gd.wait()
                    sd = pltpu.async_copy(src_h.at[pl.ds(b + CH, CH)], src_v,
                                          sem2)
                    pltpu.sync_copy(rows_v, acc.at[dst_v], add=True)
                    sd.wait()
                else:
                    pltpu.sync_copy(dst_h.at[pl.ds(b, CH)], dst_v)
                    pltpu.sync_copy(rows_v, acc.at[dst_v], add=True)
        if rem:
            rows_r, src_r, dst_r = rest
            b = base + nfull * CH
            if not constant_rows:
                pltpu.sync_copy(src_h.at[pl.ds(b, rem)], src_r)
                pltpu.async_copy(x_h.at[src_r], rows_r, sem).wait()
            pltpu.sync_copy(dst_h.at[pl.ds(b, rem)], dst_r)
            pltpu.sync_copy(rows_r, acc.at[dst_r], add=True)
        plsc.subcore_barrier()
        pltpu.sync_copy(acc.at[pl.ds(s * rpt, rpt)],
                        out_h.at[c, pl.ds(s * rpt, rpt)])

    z = jnp.zeros((CH, width), jnp.float32)
    return k(x, src, dst, z)


@functools.partial(jax.jit, static_argnames=("npad",))
def _sc_count(dst, npad):
    ones = jnp.ones((CH, 128), jnp.float32)
    return _sc_scatter_partials(ones, dst, dst, npad, 128, constant_rows=True)


@functools.partial(jax.jit, static_argnames=())
def _sc_take(x, idx):
    mp = idx.shape[0]
    rpt = mp // NT
    chunk_list = _chunks(rpt, CH)
    rem = chunk_list[-1][1] if chunk_list[-1][1] != CH else 0

    scratch = [
        pltpu.VMEM((CH, D), jnp.float32),
        pltpu.VMEM((CH,), jnp.int32),
        pltpu.SemaphoreType.DMA,
    ]
    if rem:
        scratch += [pltpu.VMEM((rem, D), jnp.float32), pltpu.VMEM((rem,), jnp.int32)]

    @functools.partial(
        pl.kernel,
        out_type=jax.ShapeDtypeStruct((mp, D), jnp.float32),
        mesh=_sc_mesh(),
        scratch_types=scratch,
    )
    def k(x_h, idx_h, out_h, rows_v, idx_v, sem, *rest):
        c = lax.axis_index("c")
        s = lax.axis_index("s")
        base = (c * NS + s) * rpt
        nfull = sum(1 for _, sz in chunk_list if sz == CH)
        if nfull:
            @pl.loop(0, nfull)
            def _(i):
                b = base + i * CH
                pltpu.sync_copy(idx_h.at[pl.ds(b, CH)], idx_v)
                pltpu.async_copy(x_h.at[idx_v], rows_v, sem).wait()
                pltpu.sync_copy(rows_v, out_h.at[pl.ds(b, CH)])
        if rem:
            rows_r, idx_r = rest
            b = base + nfull * CH
            pltpu.sync_copy(idx_h.at[pl.ds(b, rem)], idx_r)
            pltpu.async_copy(x_h.at[idx_r], rows_r, sem).wait()
            pltpu.sync_copy(rows_r, out_h.at[pl.ds(b, rem)])

    return k(x, idx)


def _blk_rows(npad):
    return 512 if npad % 512 == 0 else 320


def _tc_conv(xs, ps, cnt, w, b, relu=True):
    P = len(xs)
    npad = xs[0].shape[0]
    R = _blk_rows(npad)
    grid = (npad // R,)

    def body(*refs):
        x_refs = refs[:P]
        p_refs = refs[P:2 * P]
        cnt_ref, w_ref, b_ref, o_ref = refs[2 * P:]
        deg = jnp.sum(cnt_ref[...], axis=0)
        invd = 1.0 / jnp.maximum(deg, 1.0)
        acc = None
        for p in range(P):
            mean = jnp.sum(p_refs[p][...], axis=0) * invd
            a = x_refs[p][...] + mean
            t = jnp.dot(a, w_ref[p * 128:(p + 1) * 128, :],
                        preferred_element_type=jnp.float32)
            acc = t if acc is None else acc + t
        acc = acc + b_ref[0:1, :]
        o_ref[...] = jnp.maximum(acc, 0.0) if relu else acc

    in_specs = (
        [pl.BlockSpec((R, 128), lambda i: (i, 0)) for _ in range(P)]
        + [pl.BlockSpec((2, R, 128), lambda i: (0, i, 0)) for _ in range(P)]
        + [pl.BlockSpec((2, R, 128), lambda i: (0, i, 0)),
           pl.BlockSpec((P * 128, 128), lambda i: (0, 0)),
           pl.BlockSpec((8, 128), lambda i: (0, 0))]
    )
    return pl.pallas_call(
        body,
        grid=grid,
        in_specs=in_specs,
        out_specs=pl.BlockSpec((R, 128), lambda i: (i, 0)),
        out_shape=jax.ShapeDtypeStruct((npad, 128), jnp.float32),
    )(*xs, *ps, cnt, w, b)


def _tc_scale(p, cnt):
    npad = p.shape[1]
    R = _blk_rows(npad)
    grid = (npad // R,)

    def body(p_ref, cnt_ref, o_ref):
        s = jnp.sum(p_ref[...], axis=0)
        deg = jnp.sum(cnt_ref[...], axis=0)
        o_ref[...] = s / jnp.maximum(deg, 1.0)

    return pl.pallas_call(
        body,
        grid=grid,
        in_specs=[pl.BlockSpec((2, R, 128), lambda i: (0, i, 0)),
                  pl.BlockSpec((2, R, 128), lambda i: (0, i, 0))],
        out_specs=pl.BlockSpec((R, 128), lambda i: (i, 0)),
        out_shape=jax.ShapeDtypeStruct((npad, 128), jnp.float32),
    )(p, cnt)


def _pad_edges(e, ep, dst_pad):
    n = e.shape[1]
    src = jnp.concatenate([e[0].astype(jnp.int32),
                           jnp.zeros((ep - n,), jnp.int32)])
    dst = jnp.concatenate([e[1].astype(jnp.int32),
                           jnp.full((ep - n,), dst_pad, jnp.int32)])
    return src, dst


def _pad_idx(idx, mp, fill):
    return jnp.concatenate([idx.astype(jnp.int32),
                            jnp.full((mp - idx.shape[0],), fill, jnp.int32)])


def _pad_w(w):
    din, dout = w.shape
    return jnp.pad(w, ((0, _pad_amount(din, 128)), (0, _pad_amount(dout, 128))))


def _pad_b(b):
    bp = jnp.pad(b, (0, _pad_amount(b.shape[0], 128)))
    return jnp.broadcast_to(bp[None, :], (8, bp.shape[0]))


def _edge_pad_len(e):
    return e + _pad_amount(e, NT * 8) + CH


def _conv(x_parts, src, dst, cnt, w, b, npad):
    ps = [_sc_scatter_partials(xp, src, dst, npad, 128) for xp in x_parts]
    return _tc_conv(x_parts, ps, cnt, _pad_w(w), _pad_b(b))


def kernel(norm, geo, e0, e1, e2, cluster1, cluster2, Ws, bs):
    x = jnp.concatenate([norm, geo[:, None]], axis=1)
    x = jnp.pad(x, ((0, N0P - N0), (0, 128 - 4)))

    s0, d0 = _pad_edges(e0, _edge_pad_len(E0), N0)
    s1, d1 = _pad_edges(e1, _edge_pad_len(E1), N1)
    s2, d2 = _pad_edges(e2, _edge_pad_len(E2), N2)
    cl1 = _pad_idx(cluster1, N0P, 0)
    cl2 = _pad_idx(cluster2, N1P, 0)
    p1e = _edge_pad_len(N0)
    p2e = _edge_pad_len(N1)
    pool1_src = _pad_idx(jnp.arange(N0, dtype=jnp.int32), p1e, 0)
    pool1_dst = _pad_idx(cluster1, p1e, N1)
    pool2_src = _pad_idx(jnp.arange(N1, dtype=jnp.int32), p2e, 0)
    pool2_dst = _pad_idx(cluster2, p2e, N2)

    cnt_e0 = _sc_count(d0, N0P)
    cnt_e1 = _sc_count(d1, N1P)
    cnt_e2 = _sc_count(d2, N2P)
    cnt_c1 = _sc_count(pool1_dst, N1P)
    cnt_c2 = _sc_count(pool2_dst, N2P)

    x = _conv([x], s0, d0, cnt_e0, Ws[0], bs[0], N0P)
    x = _conv([x], s0, d0, cnt_e0, Ws[1], bs[1], N0P)
    copy0 = x
    p = _sc_scatter_partials(x, pool1_src, pool1_dst, N1P, 128)
    x = _tc_scale(p, cnt_c1)
    x = _conv([x], s1, d1, cnt_e1, Ws[2], bs[2], N1P)
    x = _conv([x], s1, d1, cnt_e1, Ws[3], bs[3], N1P)
    copy1 = x
    p = _sc_scatter_partials(x, pool2_src, pool2_dst, N2P, 128)
    x = _tc_scale(p, cnt_c2)
    x = _conv([x], s2, d2, cnt_e2, Ws[4], bs[4], N2P)
    x = _conv([x], s2, d2, cnt_e2, Ws[5], bs[5], N2P)

    up = _sc_take(x, cl2)
    x = _conv([up, copy1], s1, d1, cnt_e1, Ws[6], bs[6], N1P)
    for i in range(7, 10):
        x = _conv([x], s1, d1, cnt_e1, Ws[i], bs[i], N1P)

    up = _sc_take(x, cl1)
    x = _conv([up, copy0], s0, d0, cnt_e0, Ws[10], bs[10], N0P)
    x = _conv([x], s0, d0, cnt_e0, Ws[11], bs[11], N0P)
    x = _conv([x], s0, d0, cnt_e0, Ws[12], bs[12], N0P)
    x = _conv([x], s0, d0, cnt_e0, Ws[13], bs[13], N0P)

    return x[:N0, 0]


# speedup vs baseline: 3.3790x; 1.1055x over previous
import functools

import jax
import jax.numpy as jnp
from jax import lax
from jax.experimental import pallas as pl
from jax.experimental.pallas import tpu as pltpu
from jax.experimental.pallas import tpu_sc as plsc

N0, N1, N2 = 10000, 2500, 625
E0, E1, E2 = 320000, 80000, 20000
D = 128
NC, NS = 2, 16
NT = NC * NS
CH = 128

N0P, N1P, N2P = 10240, 2560, 640


def _pad_amount(e, m):
    return (m - e % m) % m


def _chunks(total, ch):
    out = []
    off = 0
    while off < total:
        sz = min(ch, total - off)
        out.append((off, sz))
        off += sz
    return out


def _sc_mesh():
    return plsc.VectorSubcoreMesh(core_axis_name="c", subcore_axis_name="s",
                                  num_cores=NC, num_subcores=NS)


@functools.partial(jax.jit, static_argnames=("npad", "width", "constant_rows"))
def _sc_scatter_partials(x, src, dst, npad, width, constant_rows=False):
    ep = dst.shape[0] - CH
    ept = ep // NT
    chunk_list = _chunks(ept, CH)
    rpt = npad // NS
    zero_chunks = _chunks(rpt, CH)

    scratch = [
        pltpu.VMEM_SHARED((npad, width), jnp.float32),
        pltpu.VMEM((CH, width), jnp.float32),
        pltpu.VMEM((CH, width), jnp.float32),
        pltpu.VMEM((CH,), jnp.int32),
        pltpu.VMEM((CH,), jnp.int32),
        pltpu.VMEM((CH,), jnp.int32),
        pltpu.VMEM((CH,), jnp.int32),
        pltpu.SemaphoreType.DMA,
        pltpu.SemaphoreType.DMA,
        pltpu.SemaphoreType.DMA,
    ]
    rem = chunk_list[-1][1] if chunk_list[-1][1] != CH else 0
    if rem:
        scratch += [
            pltpu.VMEM((rem, width), jnp.float32),
            pltpu.VMEM((rem,), jnp.int32),
            pltpu.VMEM((rem,), jnp.int32),
        ]

    @functools.partial(
        pl.kernel,
        out_type=jax.ShapeDtypeStruct((NC, npad, width), jnp.float32),
        mesh=_sc_mesh(),
        scratch_types=scratch,
    )
    def k(x_h, src_h, dst_h, z_h, out_h, acc, rows_v, rows_w, src_v, src_w,
          dst_v, dst_w, sem, semb, sem2, *rest):
        c = lax.axis_index("c")
        s = lax.axis_index("s")
        tid = c * NS + s
        pltpu.sync_copy(z_h, rows_v)
        for off, sz in zero_chunks:
            if sz == CH:
                pltpu.sync_copy(rows_v, acc.at[pl.ds(s * rpt + off, CH)])
            else:
                pltpu.sync_copy(rows_v.at[pl.ds(0, sz)],
                                acc.at[pl.ds(s * rpt + off, sz)])
        if constant_rows:
            pltpu.sync_copy(x_h, rows_v)
            if rem:
                pltpu.sync_copy(x_h.at[pl.ds(0, rem)], rest[0])
        plsc.subcore_barrier()

        base = tid * ept
        nfull = sum(1 for _, sz in chunk_list if sz == CH)
        npair = nfull // 2
        if nfull:
            if not constant_rows:
                pltpu.sync_copy(src_h.at[pl.ds(base, CH)], src_v)

            def single(b):
                gd = pltpu.async_copy(x_h.at[src_v], rows_v, sem)
                pltpu.sync_copy(dst_h.at[pl.ds(b, CH)], dst_v)
                gd.wait()
                sd = pltpu.async_copy(src_h.at[pl.ds(b + CH, CH)], src_v, sem2)
                pltpu.sync_copy(rows_v, acc.at[dst_v], add=True)
                sd.wait()

            if constant_rows:
                @pl.loop(0, nfull)
                def _(i):
                    b = base + i * CH
                    pltpu.sync_copy(dst_h.at[pl.ds(b, CH)], dst_v)
                    pltpu.sync_copy(rows_v, acc.at[dst_v], add=True)
            else:
                if npair:
                    @pl.loop(0, npair)
                    def _(j):
                        b = base + 2 * j * CH
                        gda = pltpu.async_copy(x_h.at[src_v], rows_v, sem)
                        pltpu.sync_copy(dst_h.at[pl.ds(b, CH)], dst_v)
                        pltpu.sync_copy(src_h.at[pl.ds(b + CH, CH)], src_w)
                        gda.wait()
                        gdb = pltpu.async_copy(x_h.at[src_w], rows_w, semb)
                        sd = pltpu.async_copy(
                            src_h.at[pl.ds(b + 2 * CH, CH)], src_v, sem2)
                        pltpu.sync_copy(dst_h.at[pl.ds(b + CH, CH)], dst_w)
                        pltpu.sync_copy(rows_v, acc.at[dst_v], add=True)
                        gdb.wait()
                        pltpu.sync_copy(rows_w, acc.at[dst_w], add=True)
                        sd.wait()
                if nfull % 2:
                    single(base + (nfull - 1) * CH)
        if rem:
            rows_r, src_r, dst_r = rest
            b = base + nfull * CH
            if not constant_rows:
                pltpu.sync_copy(src_h.at[pl.ds(b, rem)], src_r)
                pltpu.async_copy(x_h.at[src_r], rows_r, sem).wait()
            pltpu.sync_copy(dst_h.at[pl.ds(b, rem)], dst_r)
            pltpu.sync_copy(rows_r, acc.at[dst_r], add=True)
        plsc.subcore_barrier()
        pltpu.sync_copy(acc.at[pl.ds(s * rpt, rpt)],
                        out_h.at[c, pl.ds(s * rpt, rpt)])

    z = jnp.zeros((CH, width), jnp.float32)
    return k(x, src, dst, z)


@functools.partial(jax.jit, static_argnames=("npad",))
def _sc_count(dst, npad):
    ones = jnp.ones((CH, 128), jnp.float32)
    return _sc_scatter_partials(ones, dst, dst, npad, 128, constant_rows=True)


@functools.partial(jax.jit, static_argnames=())
def _sc_take(x, idx):
    mp = idx.shape[0]
    rpt = mp // NT
    chunk_list = _chunks(rpt, CH)
    rem = chunk_list[-1][1] if chunk_list[-1][1] != CH else 0

    scratch = [
        pltpu.VMEM((CH, D), jnp.float32),
        pltpu.VMEM((CH,), jnp.int32),
        pltpu.SemaphoreType.DMA,
    ]
    if rem:
        scratch += [pltpu.VMEM((rem, D), jnp.float32), pltpu.VMEM((rem,), jnp.int32)]

    @functools.partial(
        pl.kernel,
        out_type=jax.ShapeDtypeStruct((mp, D), jnp.float32),
        mesh=_sc_mesh(),
        scratch_types=scratch,
    )
    def k(x_h, idx_h, out_h, rows_v, idx_v, sem, *rest):
        c = lax.axis_index("c")
        s = lax.axis_index("s")
        base = (c * NS + s) * rpt
        nfull = sum(1 for _, sz in chunk_list if sz == CH)
        if nfull:
            @pl.loop(0, nfull)
            def _(i):
                b = base + i * CH
                pltpu.sync_copy(idx_h.at[pl.ds(b, CH)], idx_v)
                pltpu.async_copy(x_h.at[idx_v], rows_v, sem).wait()
                pltpu.sync_copy(rows_v, out_h.at[pl.ds(b, CH)])
        if rem:
            rows_r, idx_r = rest
            b = base + nfull * CH
            pltpu.sync_copy(idx_h.at[pl.ds(b, rem)], idx_r)
            pltpu.async_copy(x_h.at[idx_r], rows_r, sem).wait()
            pltpu.sync_copy(rows_r, out_h.at[pl.ds(b, rem)])

    return k(x, idx)


def _blk_rows(npad):
    return 512 if npad % 512 == 0 else 320


def _tc_conv(xs, ps, cnt, w, b, relu=True):
    P = len(xs)
    npad = xs[0].shape[0]
    R = _blk_rows(npad)
    grid = (npad // R,)

    def body(*refs):
        x_refs = refs[:P]
        p_refs = refs[P:2 * P]
        cnt_ref, w_ref, b_ref, o_ref = refs[2 * P:]
        deg = jnp.sum(cnt_ref[...], axis=0)
        invd = 1.0 / jnp.maximum(deg, 1.0)
        acc = None
        for p in range(P):
            mean = jnp.sum(p_refs[p][...], axis=0) * invd
            a = x_refs[p][...] + mean
            t = jnp.dot(a, w_ref[p * 128:(p + 1) * 128, :],
                        preferred_element_type=jnp.float32)
            acc = t if acc is None else acc + t
        acc = acc + b_ref[0:1, :]
        o_ref[...] = jnp.maximum(acc, 0.0) if relu else acc

    in_specs = (
        [pl.BlockSpec((R, 128), lambda i: (i, 0)) for _ in range(P)]
        + [pl.BlockSpec((2, R, 128), lambda i: (0, i, 0)) for _ in range(P)]
        + [pl.BlockSpec((2, R, 128), lambda i: (0, i, 0)),
           pl.BlockSpec((P * 128, 128), lambda i: (0, 0)),
           pl.BlockSpec((8, 128), lambda i: (0, 0))]
    )
    return pl.pallas_call(
        body,
        grid=grid,
        in_specs=in_specs,
        out_specs=pl.BlockSpec((R, 128), lambda i: (i, 0)),
        out_shape=jax.ShapeDtypeStruct((npad, 128), jnp.float32),
    )(*xs, *ps, cnt, w, b)


def _tc_scale(p, cnt):
    npad = p.shape[1]
    R = _blk_rows(npad)
    grid = (npad // R,)

    def body(p_ref, cnt_ref, o_ref):
        s = jnp.sum(p_ref[...], axis=0)
        deg = jnp.sum(cnt_ref[...], axis=0)
        o_ref[...] = s / jnp.maximum(deg, 1.0)

    return pl.pallas_call(
        body,
        grid=grid,
        in_specs=[pl.BlockSpec((2, R, 128), lambda i: (0, i, 0)),
                  pl.BlockSpec((2, R, 128), lambda i: (0, i, 0))],
        out_specs=pl.BlockSpec((R, 128), lambda i: (i, 0)),
        out_shape=jax.ShapeDtypeStruct((npad, 128), jnp.float32),
    )(p, cnt)


def _pad_edges(e, ep, dst_pad):
    n = e.shape[1]
    src = jnp.concatenate([e[0].astype(jnp.int32),
                           jnp.zeros((ep - n,), jnp.int32)])
    dst = jnp.concatenate([e[1].astype(jnp.int32),
                           jnp.full((ep - n,), dst_pad, jnp.int32)])
    return src, dst


def _pad_idx(idx, mp, fill):
    return jnp.concatenate([idx.astype(jnp.int32),
                            jnp.full((mp - idx.shape[0],), fill, jnp.int32)])


def _pad_w(w):
    din, dout = w.shape
    return jnp.pad(w, ((0, _pad_amount(din, 128)), (0, _pad_amount(dout, 128))))


def _pad_b(b):
    bp = jnp.pad(b, (0, _pad_amount(b.shape[0], 128)))
    return jnp.broadcast_to(bp[None, :], (8, bp.shape[0]))


def _edge_pad_len(e):
    return e + _pad_amount(e, NT * 8) + CH


def _conv(x_parts, src, dst, cnt, w, b, npad):
    ps = [_sc_scatter_partials(xp, src, dst, npad, 128) for xp in x_parts]
    return _tc_conv(x_parts, ps, cnt, _pad_w(w), _pad_b(b))


def kernel(norm, geo, e0, e1, e2, cluster1, cluster2, Ws, bs):
    x = jnp.concatenate([norm, geo[:, None]], axis=1)
    x = jnp.pad(x, ((0, N0P - N0), (0, 128 - 4)))

    s0, d0 = _pad_edges(e0, _edge_pad_len(E0), N0)
    s1, d1 = _pad_edges(e1, _edge_pad_len(E1), N1)
    s2, d2 = _pad_edges(e2, _edge_pad_len(E2), N2)
    cl1 = _pad_idx(cluster1, N0P, 0)
    cl2 = _pad_idx(cluster2, N1P, 0)
    p1e = _edge_pad_len(N0)
    p2e = _edge_pad_len(N1)
    pool1_src = _pad_idx(jnp.arange(N0, dtype=jnp.int32), p1e, 0)
    pool1_dst = _pad_idx(cluster1, p1e, N1)
    pool2_src = _pad_idx(jnp.arange(N1, dtype=jnp.int32), p2e, 0)
    pool2_dst = _pad_idx(cluster2, p2e, N2)

    cnt_e0 = _sc_count(d0, N0P)
    cnt_e1 = _sc_count(d1, N1P)
    cnt_e2 = _sc_count(d2, N2P)
    cnt_c1 = _sc_count(pool1_dst, N1P)
    cnt_c2 = _sc_count(pool2_dst, N2P)

    x = _conv([x], s0, d0, cnt_e0, Ws[0], bs[0], N0P)
    x = _conv([x], s0, d0, cnt_e0, Ws[1], bs[1], N0P)
    copy0 = x
    p = _sc_scatter_partials(x, pool1_src, pool1_dst, N1P, 128)
    x = _tc_scale(p, cnt_c1)
    x = _conv([x], s1, d1, cnt_e1, Ws[2], bs[2], N1P)
    x = _conv([x], s1, d1, cnt_e1, Ws[3], bs[3], N1P)
    copy1 = x
    p = _sc_scatter_partials(x, pool2_src, pool2_dst, N2P, 128)
    x = _tc_scale(p, cnt_c2)
    x = _conv([x], s2, d2, cnt_e2, Ws[4], bs[4], N2P)
    x = _conv([x], s2, d2, cnt_e2, Ws[5], bs[5], N2P)

    up = _sc_take(x, cl2)
    x = _conv([up, copy1], s1, d1, cnt_e1, Ws[6], bs[6], N1P)
    for i in range(7, 10):
        x = _conv([x], s1, d1, cnt_e1, Ws[i], bs[i], N1P)

    up = _sc_take(x, cl1)
    x = _conv([up, copy0], s0, d0, cnt_e0, Ws[10], bs[10], N0P)
    x = _conv([x], s0, d0, cnt_e0, Ws[11], bs[11], N0P)
    x = _conv([x], s0, d0, cnt_e0, Ws[12], bs[12], N0P)
    x = _conv([x], s0, d0, cnt_e0, Ws[13], bs[13], N0P)

    return x[:N0, 0]
